# baseline (speedup 1.0000x reference)
import functools

import jax
import jax.numpy as jnp
from jax import lax
from jax.experimental import pallas as pl
from jax.experimental.pallas import tpu as pltpu
from jax.experimental.pallas import tpu_sc as plsc

_CHUNK = 128


def _sc_gather4(user_idx, item_idx, t_umf, t_imf, t_umlp, t_imlp):
    B = user_idx.shape[0]
    D = t_umf.shape[1]
    info = plsc.get_sparse_core_info()
    nw = info.num_cores * info.num_subcores
    bpw = B // nw
    nch = bpw // _CHUNK
    nc = info.num_cores
    mesh = plsc.VectorSubcoreMesh(core_axis_name="c", subcore_axis_name="s")
    row_t = jax.ShapeDtypeStruct((B, D), jnp.float32)

    @functools.partial(
        pl.kernel,
        mesh=mesh,
        out_type=[row_t, row_t, row_t, row_t],
        compiler_params=pltpu.CompilerParams(use_tc_tiling_on_sc=False),
        scratch_types=[
            pltpu.VMEM((bpw,), jnp.int32),
            pltpu.VMEM((bpw,), jnp.int32),
            pltpu.VMEM((bpw, D), jnp.float32),
            pltpu.VMEM((bpw, D), jnp.float32),
            pltpu.VMEM((bpw, D), jnp.float32),
            pltpu.VMEM((bpw, D), jnp.float32),
            pltpu.SemaphoreType.DMA,
        ],
    )
    def gather_kernel(uidx, iidx, umf, imf, umlp, imlp,
                      o_umf, o_imf, o_umlp, o_imlp,
                      idx_u, idx_i, r0, r1, r2, r3, sem):
        wid = lax.axis_index("s") * nc + lax.axis_index("c")
        base = wid * bpw
        pltpu.sync_copy(uidx.at[pl.ds(base, bpw)], idx_u)
        pltpu.sync_copy(iidx.at[pl.ds(base, bpw)], idx_i)
        copies = []
        for tbl, idx, dst in ((umf, idx_u, r0), (imf, idx_i, r1),
                              (umlp, idx_u, r2), (imlp, idx_i, r3)):
            for c in range(nch):
                sl = pl.ds(c * _CHUNK, _CHUNK)
                copies.append(pltpu.async_copy(tbl.at[idx.at[sl]], dst.at[sl], sem))
        for cp in copies:
            cp.wait()
        osl = pl.ds(base, bpw)
        pltpu.sync_copy(r0, o_umf.at[osl])
        pltpu.sync_copy(r1, o_imf.at[osl])
        pltpu.sync_copy(r2, o_umlp.at[osl])
        pltpu.sync_copy(r3, o_imlp.at[osl])

    return gather_kernel(user_idx, item_idx, t_umf, t_imf, t_umlp, t_imlp)


def _tc_mlp(u_mf, i_mf, u_mlp, i_mlp, W1, b1, W2, b2, W3, b3, Wo, bo):
    B, D = u_mf.shape
    BLK = 2048
    grid = B // BLK
    w1a, w1b = W1[:D], W1[D:]
    womf_t = Wo[:D].reshape(1, D)
    woh_t = Wo[D:].reshape(1, -1)
    b1r = b1.reshape(1, -1)
    b2r = b2.reshape(1, -1)
    b3r = b3.reshape(1, -1)
    bor = bo.reshape(1, 1)

    def body(umf_ref, imf_ref, umlp_ref, imlp_ref,
             w1a_ref, w1b_ref, b1_ref, w2_ref, b2_ref, w3_ref, b3_ref,
             womf_ref, woh_ref, bo_ref, out_ref):
        h = jnp.dot(umlp_ref[...], w1a_ref[...], preferred_element_type=jnp.float32)
        h = h + jnp.dot(imlp_ref[...], w1b_ref[...], preferred_element_type=jnp.float32)
        h = jnp.maximum(h + b1_ref[...], 0.0)
        h = jnp.dot(h, w2_ref[...], preferred_element_type=jnp.float32)
        h = jnp.maximum(h + b2_ref[...], 0.0)
        h = jnp.dot(h, w3_ref[...], preferred_element_type=jnp.float32)
        h = jnp.maximum(h + b3_ref[...], 0.0)
        mf = umf_ref[...] * imf_ref[...]
        acc = mf * womf_ref[...] + h * woh_ref[...]
        out_ref[...] = jnp.sum(acc, axis=1, keepdims=True) + bo_ref[...]

    row_spec = pl.BlockSpec((BLK, D), lambda i: (i, 0))
    full = lambda a: pl.BlockSpec(a.shape, lambda i: (0,) * a.ndim)
    out = pl.pallas_call(
        body,
        grid=(grid,),
        in_specs=[row_spec, row_spec, row_spec, row_spec,
                  full(w1a), full(w1b), full(b1r), full(W2), full(b2r),
                  full(W3), full(b3r), full(womf_t), full(woh_t), full(bor)],
        out_specs=pl.BlockSpec((BLK, 1), lambda i: (i, 0)),
        out_shape=jax.ShapeDtypeStruct((B, 1), jnp.float32),
    )(u_mf, i_mf, u_mlp, i_mlp, w1a, w1b, b1r, W2, b2r, W3, b3r,
      womf_t, woh_t, bor)
    return out[:, 0]


def kernel(user_idx, item_idx, user_embedding_mf, item_embedding_mf,
           user_embedding_mlp, item_embedding_mlp, W1, b1, W2, b2, W3, b3,
           Wo, bo):
    u_mf, i_mf, u_mlp, i_mlp = _sc_gather4(
        user_idx.astype(jnp.int32), item_idx.astype(jnp.int32),
        user_embedding_mf, item_embedding_mf,
        user_embedding_mlp, item_embedding_mlp)
    return _tc_mlp(u_mf, i_mf, u_mlp, i_mlp, W1, b1, W2, b2, W3, b3, Wo, bo)


# speedup vs baseline: 1.4140x; 1.4140x over previous
import functools

import jax
import jax.numpy as jnp
from jax import lax
from jax.experimental import pallas as pl
from jax.experimental.pallas import tpu as pltpu
from jax.experimental.pallas import tpu_sc as plsc


def _sc_gather4(uidx, iidx, t_umf, t_imf, t_umlp, t_imlp):
    B = uidx.shape[0]
    D = t_umf.shape[1]
    info = plsc.get_sparse_core_info()
    nc = info.num_cores
    nw = nc * info.num_subcores
    bpw = B // nw
    mesh = plsc.VectorSubcoreMesh(core_axis_name="c", subcore_axis_name="s")
    out_t = jax.ShapeDtypeStruct((B, D), jnp.float32)

    @functools.partial(
        pl.kernel,
        mesh=mesh,
        out_type=[out_t, out_t, out_t, out_t],
        scratch_types=[
            pltpu.VMEM((bpw,), jnp.int32),
            pltpu.VMEM((bpw,), jnp.int32),
            pltpu.VMEM((bpw, D), jnp.float32),
            pltpu.SemaphoreType.DMA,
            pltpu.SemaphoreType.DMA,
        ],
    )
    def gather_kernel(uidx_hbm, iidx_hbm, umf, imf, umlp, imlp,
                      o_umf, o_imf, o_umlp, o_imlp,
                      sidx_u, sidx_i, rows, gsem, wsem):
        wid = lax.axis_index("s") * nc + lax.axis_index("c")
        base = wid * bpw
        sl = pl.ds(base, bpw)
        pltpu.sync_copy(uidx_hbm.at[sl], sidx_u)
        pltpu.sync_copy(iidx_hbm.at[sl], sidx_i)

        def do_table(ti, tbl, sidx, oref):
            if ti > 0:
                pltpu.make_async_copy(rows, oref.at[sl], wsem).wait()

            def fire(j, _):
                v = sidx[pl.ds(j * 16, 16)]
                for l in range(16):
                    pltpu.async_copy(tbl.at[v[l]], rows.at[j * 16 + l], gsem)
                return _

            lax.fori_loop(0, bpw // 16, fire, None)

            def drain(j, _):
                pltpu.make_async_copy(tbl.at[0], rows.at[0], gsem).wait()
                return _

            lax.fori_loop(0, bpw, drain, None)
            pltpu.async_copy(rows, oref.at[sl], wsem)

        do_table(0, umf, sidx_u, o_umf)
        do_table(1, imf, sidx_i, o_imf)
        do_table(2, umlp, sidx_u, o_umlp)
        do_table(3, imlp, sidx_i, o_imlp)
        pltpu.make_async_copy(rows, o_imlp.at[sl], wsem).wait()

    return gather_kernel(uidx, iidx, t_umf, t_imf, t_umlp, t_imlp)


def _tc_mlp(u_mf, i_mf, u_mlp, i_mlp, W1, b1, W2, b2, W3, b3, Wo, bo):
    B, D = u_mf.shape
    BLK = 2048
    grid = B // BLK
    w1a, w1b = W1[:D], W1[D:]
    womf_t = Wo[:D].reshape(1, D)
    woh_t = Wo[D:].reshape(1, -1)
    b1r = b1.reshape(1, -1)
    b2r = b2.reshape(1, -1)
    b3r = b3.reshape(1, -1)
    bor = bo.reshape(1, 1)

    def body(umf_ref, imf_ref, umlp_ref, imlp_ref,
             w1a_ref, w1b_ref, b1_ref, w2_ref, b2_ref, w3_ref, b3_ref,
             womf_ref, woh_ref, bo_ref, out_ref):
        h = jnp.dot(umlp_ref[...], w1a_ref[...],
                    preferred_element_type=jnp.float32)
        h = h + jnp.dot(imlp_ref[...], w1b_ref[...],
                        preferred_element_type=jnp.float32)
        h = jnp.maximum(h + b1_ref[...], 0.0)
        h = jnp.dot(h, w2_ref[...], preferred_element_type=jnp.float32)
        h = jnp.maximum(h + b2_ref[...], 0.0)
        h = jnp.dot(h, w3_ref[...], preferred_element_type=jnp.float32)
        h = jnp.maximum(h + b3_ref[...], 0.0)
        mf = umf_ref[...] * imf_ref[...]
        acc = mf * womf_ref[...] + h * woh_ref[...]
        out_ref[...] = jnp.sum(acc, axis=1, keepdims=True) + bo_ref[...]

    row_spec = pl.BlockSpec((BLK, D), lambda i: (i, 0))
    full = lambda a: pl.BlockSpec(a.shape, lambda i: (0,) * a.ndim)
    out = pl.pallas_call(
        body,
        grid=(grid,),
        in_specs=[row_spec, row_spec, row_spec, row_spec,
                  full(w1a), full(w1b), full(b1r), full(W2), full(b2r),
                  full(W3), full(b3r), full(womf_t), full(woh_t), full(bor)],
        out_specs=pl.BlockSpec((BLK, 1), lambda i: (i, 0)),
        out_shape=jax.ShapeDtypeStruct((B, 1), jnp.float32),
    )(u_mf, i_mf, u_mlp, i_mlp, w1a, w1b, b1r, W2, b2r, W3, b3r,
      womf_t, woh_t, bor)
    return out[:, 0]


def kernel(user_idx, item_idx, user_embedding_mf, item_embedding_mf,
           user_embedding_mlp, item_embedding_mlp, W1, b1, W2, b2, W3, b3,
           Wo, bo):
    u_mf, i_mf, u_mlp, i_mlp = _sc_gather4(
        user_idx.astype(jnp.int32), item_idx.astype(jnp.int32),
        user_embedding_mf, item_embedding_mf,
        user_embedding_mlp, item_embedding_mlp)
    return _tc_mlp(u_mf, i_mf, u_mlp, i_mlp, W1, b1, W2, b2, W3, b3, Wo, bo)
